# baseline (speedup 1.0000x reference)
import functools

import jax
import jax.numpy as jnp
from jax import lax
from jax.experimental import pallas as pl
from jax.experimental.pallas import tpu as pltpu
from jax.experimental.pallas import tpu_sc as plsc

BATCH = 4096
SEQ = 200
D = 64

_info = plsc.get_sparse_core_info()
_NC, _NS = _info.num_cores, _info.num_subcores
NW = _NC * _NS
ROWS_PER_W = BATCH // NW

_mesh = plsc.VectorSubcoreMesh(core_axis_name="c", subcore_axis_name="s")


@functools.partial(
    pl.kernel,
    out_type=jax.ShapeDtypeStruct((BATCH, SEQ, D), jnp.float32),
    mesh=_mesh,
    scratch_types=[
        pltpu.VMEM((ROWS_PER_W, SEQ), jnp.int32),
        pltpu.VMEM_SHARED((1001, D), jnp.float32),
        pltpu.VMEM((SEQ, D), jnp.float32),
        pltpu.VMEM((SEQ, D), jnp.float32),
        pltpu.SemaphoreType.DMA,
        pltpu.SemaphoreType.DMA,
        pltpu.SemaphoreType.DMA,
        pltpu.SemaphoreType.DMA,
    ],
    compiler_params=pltpu.CompilerParams(use_tc_tiling_on_sc=False),
)
def _embed(
    idx_hbm, table_hbm, out_hbm, idx_v, table_v, rows0, rows1, sg0, sg1, sw0, sw1
):
    wid = lax.axis_index("s") * _NC + lax.axis_index("c")
    base = wid * ROWS_PER_W
    rows = (rows0, rows1)
    sg = (sg0, sg1)
    sw = (sw0, sw1)

    pltpu.sync_copy(idx_hbm.at[pl.ds(base, ROWS_PER_W), :], idx_v)

    @pl.when(lax.axis_index("s") == 0)
    def _load_table():
        pltpu.sync_copy(table_hbm, table_v)

    plsc.subcore_barrier()

    def gather_start(i, b):
        pltpu.async_copy(table_v.at[idx_v.at[i]], rows[b], sg[b])

    def gather_wait(i, b):
        pltpu.make_async_copy(table_v.at[idx_v.at[i]], rows[b], sg[b]).wait()

    def wb_start(i, b):
        pltpu.async_copy(rows[b], out_hbm.at[base + i], sw[b])

    def wb_wait(i, b):
        pltpu.make_async_copy(rows[b], out_hbm.at[base + i], sw[b]).wait()

    gather_start(0, 0)
    gather_start(1, 1)
    gather_wait(0, 0)
    wb_start(0, 0)
    gather_wait(1, 1)
    wb_start(1, 1)

    def body(g, carry):
        for b in range(2):
            i = 2 * g + b
            wb_wait(i - 2, b)
            gather_start(i, b)
            gather_wait(i, b)
            wb_start(i, b)
        return carry

    lax.fori_loop(1, ROWS_PER_W // 2, body, 0)

    wb_wait(ROWS_PER_W - 2, 0)
    wb_wait(ROWS_PER_W - 1, 1)


def kernel(word_sequences, embedding_weight):
    return _embed(word_sequences, embedding_weight)
